# Initial kernel scaffold; baseline (speedup 1.0000x reference)
import functools

import jax
import jax.numpy as jnp
from jax import lax
from jax.experimental import pallas as pl
from jax.experimental.pallas import tpu as pltpu
from jax.experimental.pallas import tpu_sc as plsc

IN_F = 65536
OUT_F = 16384
NCON = 64
NC = 2
NS = 16
NW = NC * NS
ROWS_PER_W = OUT_F // NW
GROUPS = ROWS_PER_W // 16


def _body(x_hbm, conn_hbm, op_hbm, out_hbm, x_v, conn_v, op_v, out_v):
    wid = lax.axis_index("s") * NC + lax.axis_index("c")
    base = wid * ROWS_PER_W

    pltpu.sync_copy(x_hbm, x_v)
    pltpu.sync_copy(conn_hbm.at[pl.ds(base * NCON, ROWS_PER_W * NCON)], conn_v)
    pltpu.sync_copy(op_hbm.at[pl.ds(base, ROWS_PER_W)], op_v)

    lane = lax.iota(jnp.int32, (16,))
    row_off = lane * NCON

    def group(g, carry):
        pos0 = (g * 16) * NCON + row_off

        def conn_step(j, mm):
            mins, maxs = mm
            ci = plsc.load_gather(conn_v, [pos0 + j])
            v = plsc.load_gather(x_v, [ci])
            return jnp.minimum(mins, v), jnp.maximum(maxs, v)

        inf = jnp.full((16,), jnp.inf, jnp.float32)
        mins, maxs = lax.fori_loop(0, NCON, conn_step, (inf, -inf), unroll=8)
        opv = op_v[pl.ds(g * 16, 16)]
        out_v[pl.ds(g * 16, 16)] = jnp.where(opv == 0, mins, maxs)
        return carry

    lax.fori_loop(0, GROUPS, group, 0)
    pltpu.sync_copy(out_v, out_hbm.at[pl.ds(base, ROWS_PER_W)])


@jax.jit
def kernel(x, connection_indices, operator_indices):
    conn = connection_indices.reshape(-1).astype(jnp.int32)
    op = operator_indices.astype(jnp.int32)

    mesh = plsc.VectorSubcoreMesh(core_axis_name="c", subcore_axis_name="s")
    call = functools.partial(
        pl.kernel,
        mesh=mesh,
        out_type=jax.ShapeDtypeStruct((OUT_F,), jnp.float32),
        scratch_types=[
            pltpu.VMEM((IN_F,), jnp.float32),
            pltpu.VMEM((ROWS_PER_W * NCON,), jnp.int32),
            pltpu.VMEM((ROWS_PER_W,), jnp.int32),
            pltpu.VMEM((ROWS_PER_W,), jnp.float32),
        ],
    )(_body)
    return call(x, conn, op)


# speedup vs baseline: 100.6479x; 100.6479x over previous
import functools

import jax
import jax.numpy as jnp
from jax import lax
from jax.experimental import pallas as pl
from jax.experimental.pallas import tpu as pltpu
from jax.experimental.pallas import tpu_sc as plsc

IN_F = 65536
OUT_F = 16384
NCON = 64
NC = 2
NS = 16
NW = NC * NS
ROWS_PER_W = OUT_F // NW
GROUPS = ROWS_PER_W // 16


def _body(x_hbm, conn_hbm, op_hbm, out_hbm, x_v, conn_v, op_v, out_v):
    wid = lax.axis_index("s") * NC + lax.axis_index("c")
    base = wid * ROWS_PER_W

    pltpu.sync_copy(x_hbm, x_v)
    pltpu.sync_copy(conn_hbm.at[pl.ds(base * NCON, ROWS_PER_W * NCON)], conn_v)
    pltpu.sync_copy(op_hbm.at[pl.ds(base, ROWS_PER_W)], op_v)

    lane = lax.iota(jnp.int32, 16)
    row_off = lane * NCON

    def group(g, carry):
        pos0 = (g * 16) * NCON + row_off

        def conn_step(j, mm):
            mins, maxs = mm
            ci = plsc.load_gather(conn_v, [pos0 + j])
            v = plsc.load_gather(x_v, [ci])
            return jnp.minimum(mins, v), jnp.maximum(maxs, v)

        inf = jnp.full((16,), jnp.inf, jnp.float32)
        mins, maxs = lax.fori_loop(0, NCON, conn_step, (inf, -inf), unroll=8)
        opv = op_v[pl.ds(g * 16, 16)]
        out_v[pl.ds(g * 16, 16)] = jnp.where(opv == 0, mins, maxs)
        return carry

    lax.fori_loop(0, GROUPS, group, 0)
    pltpu.sync_copy(out_v, out_hbm.at[pl.ds(base, ROWS_PER_W)])


@jax.jit
def kernel(x, connection_indices, operator_indices):
    conn = connection_indices.reshape(-1).astype(jnp.int32)
    op = operator_indices.astype(jnp.int32)

    mesh = plsc.VectorSubcoreMesh(core_axis_name="c", subcore_axis_name="s")
    call = functools.partial(
        pl.kernel,
        mesh=mesh,
        out_type=jax.ShapeDtypeStruct((OUT_F,), jnp.float32),
        compiler_params=pltpu.CompilerParams(needs_layout_passes=False),
        scratch_types=[
            pltpu.VMEM((IN_F,), jnp.float32),
            pltpu.VMEM((ROWS_PER_W * NCON,), jnp.int32),
            pltpu.VMEM((ROWS_PER_W,), jnp.int32),
            pltpu.VMEM((ROWS_PER_W,), jnp.float32),
        ],
    )(_body)
    return call(x, conn, op)
